# Initial kernel scaffold; baseline (speedup 1.0000x reference)
import functools

import jax
import jax.numpy as jnp
from jax import lax
from jax.experimental import pallas as pl
from jax.experimental.pallas import tpu as pltpu
from jax.experimental.pallas import tpu_sc as plsc

N = 10000
E = 320000
G = 64
K = 30
NPAD = 10240
TRASH = N
DEGW = 16
CHUNK = 128
NCHUNKS = E // CHUNK
NC = 2
NS = 16
NW = NC * NS
RPT = NPAD // NS

_MESH = plsc.VectorSubcoreMesh(
    core_axis_name="c", subcore_axis_name="s", num_cores=NC, num_subcores=NS)


def _worker_ids():
    cid = lax.axis_index("c")
    sid = lax.axis_index("s")
    return cid, sid, sid * NC + cid


def _zero_tile_rows(zbuf, shared, sid, width):
    def zrow(i, _):
        for c0 in range(0, width, 16):
            zbuf[i, pl.ds(c0, 16)] = jnp.zeros((16,), jnp.float32)
        return 0
    lax.fori_loop(0, RPT, zrow, 0)
    pltpu.sync_copy(zbuf, shared.at[pl.ds(sid * RPT, RPT)])


def _edge_loop(wid, body):
    n_my = (NCHUNKS - wid + NW - 1) // NW

    def step(t, _):
        body(wid + t * NW)
        return 0
    lax.fori_loop(0, n_my, step, 0)


@functools.partial(
    pl.kernel,
    out_type=(jax.ShapeDtypeStruct((E,), jnp.int32),
              jax.ShapeDtypeStruct((NC, NPAD, DEGW), jnp.float32)),
    mesh=_MESH,
    scratch_types=[
        pltpu.VMEM((CHUNK,), jnp.int32),
        pltpu.VMEM((CHUNK,), jnp.int32),
        pltpu.VMEM((CHUNK,), jnp.int32),
        pltpu.VMEM((CHUNK, DEGW), jnp.float32),
        pltpu.VMEM((RPT, DEGW), jnp.float32),
        pltpu.VMEM_SHARED((NPAD, DEGW), jnp.float32),
    ],
)
def _sc_prep(row_hbm, col_hbm, colp_hbm, degp_hbm,
             r_v, c_v, cp_v, ones_v, zbuf, deg_sh):
    cid, sid, wid = _worker_ids()

    def fill_ones(i, _):
        ones_v[i, :] = jnp.ones((16,), jnp.float32)
        return 0
    lax.fori_loop(0, CHUNK, fill_ones, 0)
    _zero_tile_rows(zbuf, deg_sh, sid, DEGW)
    plsc.subcore_barrier()

    def do_chunk(j):
        base = j * CHUNK
        pltpu.sync_copy(row_hbm.at[pl.ds(base, CHUNK)], r_v)
        pltpu.sync_copy(col_hbm.at[pl.ds(base, CHUNK)], c_v)
        for i in range(0, CHUNK, 16):
            r16 = r_v[pl.ds(i, 16)]
            c16 = c_v[pl.ds(i, 16)]
            cp_v[pl.ds(i, 16)] = jnp.where(
                r16 != c16, c16, jnp.full((16,), TRASH, jnp.int32))
        pltpu.sync_copy(cp_v, colp_hbm.at[pl.ds(base, CHUNK)])
        pltpu.sync_copy(ones_v, deg_sh.at[cp_v], add=True)

    _edge_loop(wid, do_chunk)
    plsc.subcore_barrier()
    pltpu.sync_copy(deg_sh.at[pl.ds(sid * RPT, RPT)],
                    degp_hbm.at[cid, pl.ds(sid * RPT, RPT)])


def _make_sc_scatter(fw):
    @functools.partial(
        pl.kernel,
        out_type=jax.ShapeDtypeStruct((NC, NPAD, fw), jnp.float32),
        mesh=_MESH,
        scratch_types=[
            pltpu.VMEM((CHUNK,), jnp.int32),
            pltpu.VMEM((CHUNK,), jnp.int32),
            pltpu.VMEM((CHUNK, fw), jnp.float32),
            pltpu.VMEM((RPT, fw), jnp.float32),
            pltpu.VMEM_SHARED((NPAD, fw), jnp.float32),
            pltpu.SemaphoreType.DMA,
        ],
    )
    def sc_scatter(g_hbm, row_hbm, colp_hbm, sp_hbm,
                   ridx_v, cidx_v, rows_v, zbuf, acc_sh, sem):
        cid, sid, wid = _worker_ids()
        _zero_tile_rows(zbuf, acc_sh, sid, fw)
        plsc.subcore_barrier()

        def do_chunk(j):
            base = j * CHUNK
            pltpu.sync_copy(row_hbm.at[pl.ds(base, CHUNK)], ridx_v)
            pltpu.sync_copy(colp_hbm.at[pl.ds(base, CHUNK)], cidx_v)
            pltpu.async_copy(g_hbm.at[ridx_v], rows_v, sem).wait()
            pltpu.sync_copy(rows_v, acc_sh.at[cidx_v], add=True)

        _edge_loop(wid, do_chunk)
        plsc.subcore_barrier()
        pltpu.sync_copy(acc_sh.at[pl.ds(sid * RPT, RPT)],
                        sp_hbm.at[cid, pl.ds(sid * RPT, RPT)])

    return sc_scatter


_sc_scatter32 = _make_sc_scatter(32)
_sc_scatter16 = _make_sc_scatter(16)


NPOOL = G * 32
PPW = NPOOL // NW


@functools.partial(
    pl.kernel,
    out_type=jax.ShapeDtypeStruct((NPOOL, 128), jnp.float32),
    mesh=_MESH,
    scratch_types=[
        pltpu.VMEM((PPW,), jnp.int32),
        pltpu.VMEM((PPW, 128), jnp.float32),
        pltpu.SemaphoreType.DMA,
    ],
)
def _sc_pool_gather(xc_hbm, idx_hbm, pooled_hbm, idx_v, rows_v, sem):
    _, _, wid = _worker_ids()
    base = wid * PPW
    pltpu.sync_copy(idx_hbm.at[pl.ds(base, PPW)], idx_v)
    pltpu.async_copy(xc_hbm.at[idx_v], rows_v, sem).wait()
    pltpu.sync_copy(rows_v, pooled_hbm.at[pl.ds(base, PPW)])


def _tc_first(x_ref, w1_ref, degp_ref, dinv_ref, g1_ref):
    deg = degp_ref[0, :, 0:1] + degp_ref[1, :, 0:1]
    dinv = lax.rsqrt(deg[0:N] + 1.0)
    dinv_ref[...] = dinv
    h = jnp.dot(x_ref[...], w1_ref[...], preferred_element_type=jnp.float32)
    g1_ref[...] = h * dinv


def _tc_combine(sp_ref, g_ref, dinv_ref, b_ref, wn_ref, xl_ref, gn_ref):
    s = sp_ref[0, 0:N, :] + sp_ref[1, 0:N, :]
    dinv = dinv_ref[...]
    xl = jnp.tanh(dinv * (s + g_ref[...]) + b_ref[...])
    xl_ref[...] = xl
    h = jnp.dot(xl, wn_ref[...], preferred_element_type=jnp.float32)
    gn_ref[...] = h * dinv


def _tc_last(sp_ref, g_ref, dinv_ref, b_ref, x1_ref, x2_ref, x3_ref,
             xc_ref, key_ref):
    s = sp_ref[0, 0:N, :] + sp_ref[1, 0:N, :]
    x4 = jnp.tanh(dinv_ref[...] * (s + g_ref[...]) + b_ref[...])
    xc = jnp.concatenate(
        [x1_ref[...], x2_ref[...], x3_ref[...], x4,
         jnp.zeros((N, 16), jnp.float32)], axis=1)
    xc_ref[0:N, :] = xc
    xc_ref[N:NPAD, :] = jnp.zeros((NPAD - N, 128), jnp.float32)
    key_ref[0:N, :] = x4[:, 0:1]
    key_ref[N:NPAD, :] = jnp.full((NPAD - N, 1), -2.0, jnp.float32)


def _tc_topk(key_ref, batch_ref, idx_ref):
    key0 = key_ref[...]
    batch2d = batch_ref[...]
    gids = lax.broadcasted_iota(jnp.int32, (G, 1, 1), 0)
    gmask = batch2d[None, :, :] == gids
    iota_n = (lax.broadcasted_iota(jnp.int32, (NPAD // 128, 128), 0) * 128
              + lax.broadcasted_iota(jnp.int32, (NPAD // 128, 128), 1))
    lane32 = lax.broadcasted_iota(jnp.int32, (G, 32), 1)
    big = jnp.int32(2**30)

    def step(j, carry):
        key, idxmat = carry
        kv = jnp.where(gmask, key[None, :, :], -2.0)
        maxv = jnp.max(kv, axis=(1, 2))
        eq = kv == maxv[:, None, None]
        idxv = jnp.min(jnp.where(eq, iota_n[None, :, :], big), axis=(1, 2))
        idxv = jnp.where(maxv < -1.5, TRASH, idxv)
        idxmat = jnp.where(lane32 == j, idxv[:, None], idxmat)
        sel = jnp.any(iota_n[None, :, :] == idxv[:, None, None], axis=0)
        key = jnp.where(sel, -2.0, key)
        return key, idxmat

    idx0 = jnp.full((G, 32), TRASH, jnp.int32)
    _, idxmat = lax.fori_loop(0, K, step, (key0, idx0))
    idx_ref[...] = idxmat


def _tc_head(pooled_ref, wp_ref, fc1w_ref, fc1b_ref, fc2w_ref, fc2b_ref,
             out_ref):
    P = jnp.dot(pooled_ref[...], wp_ref[...],
                preferred_element_type=jnp.float32)
    bd = (lax.broadcasted_iota(jnp.int32, (512, 16), 0) // 32
          == lax.broadcasted_iota(jnp.int32, (512, 16), 1)
          ).astype(jnp.float32)
    n1 = jnp.dot(P * P, bd, preferred_element_type=jnp.float32)
    slot_ok = (lax.broadcasted_iota(jnp.int32, (NPOOL, 16), 0) % 32) < K
    out = jnp.sum(P.reshape(G, 32, 512), axis=1) * (1.0 / K)

    for _ in range(3):
        orows = jnp.broadcast_to(
            out.reshape(G, 1, 512), (G, 32, 512)).reshape(NPOOL, 512)
        dot = jnp.dot(P * orows, bd, preferred_element_type=jnp.float32)
        n2 = jnp.dot(out * out, bd, preferred_element_type=jnp.float32)
        n2r = jnp.broadcast_to(
            n2.reshape(G, 1, 16), (G, 32, 16)).reshape(NPOOL, 16)
        logits = dot / jnp.maximum(n1 + n2r - dot, 1e-8)
        logits = jnp.where(slot_ok, logits, -1e30)
        l3 = logits.reshape(G, 32, 16)
        m = jnp.max(l3, axis=1, keepdims=True)
        e = jnp.exp(l3 - m)
        coef = (e / jnp.sum(e, axis=1, keepdims=True)).reshape(NPOOL, 16)
        cexp = jnp.broadcast_to(
            coef.reshape(NPOOL, 16, 1), (NPOOL, 16, 32)).reshape(NPOOL, 512)
        out = jnp.sum((cexp * P).reshape(G, 32, 512), axis=1)

    n2f = jnp.dot(out * out, bd, preferred_element_type=jnp.float32)
    scale = (n2f / (1.0 + n2f)) / jnp.sqrt(n2f + 1e-8)
    sexp = jnp.broadcast_to(
        scale.reshape(G, 16, 1), (G, 16, 32)).reshape(G, 512)
    caps = out * sexp
    h = jnp.maximum(
        jnp.dot(caps, fc1w_ref[...], preferred_element_type=jnp.float32)
        + fc1b_ref[...], 0.0)
    z = (jnp.dot(h, fc2w_ref[...], preferred_element_type=jnp.float32)
         + fc2b_ref[...])
    m = jnp.max(z, axis=1, keepdims=True)
    lse = m + jnp.log(jnp.sum(jnp.exp(z - m), axis=1, keepdims=True))
    out_ref[...] = z - lse


def _tc_call(body, out_shapes):
    return pl.pallas_call(body, out_shape=out_shapes)


def kernel(x, edge_index, batch, W1, b1, W2, b2, W3, b3, W4, b4,
           caps_W, fc1_W, fc1_b, fc2_W, fc2_b):
    f32 = jnp.float32
    row = edge_index[0]
    col = edge_index[1]

    colp, degp = _sc_prep(row, col)

    dinv, g1 = _tc_call(_tc_first, (
        jax.ShapeDtypeStruct((N, 1), f32),
        jax.ShapeDtypeStruct((N, 32), f32),
    ))(x, W1, degp)

    sp1 = _sc_scatter32(g1, row, colp)
    x1, g2 = _tc_call(_tc_combine, (
        jax.ShapeDtypeStruct((N, 32), f32),
        jax.ShapeDtypeStruct((N, 32), f32),
    ))(sp1, g1, dinv, b1.reshape(1, 32), W2)

    sp2 = _sc_scatter32(g2, row, colp)
    x2, g3 = _tc_call(_tc_combine, (
        jax.ShapeDtypeStruct((N, 32), f32),
        jax.ShapeDtypeStruct((N, 32), f32),
    ))(sp2, g2, dinv, b2.reshape(1, 32), W3)

    sp3 = _sc_scatter32(g3, row, colp)
    W4p = jnp.pad(W4, ((0, 0), (0, 15)))
    x3, g4 = _tc_call(_tc_combine, (
        jax.ShapeDtypeStruct((N, 32), f32),
        jax.ShapeDtypeStruct((N, 16), f32),
    ))(sp3, g3, dinv, b3.reshape(1, 32), W4p)

    sp4 = _sc_scatter16(g4, row, colp)
    b4p = jnp.pad(b4, (0, 15)).reshape(1, 16)
    xc, keycol = _tc_call(_tc_last, (
        jax.ShapeDtypeStruct((NPAD, 128), f32),
        jax.ShapeDtypeStruct((NPAD, 1), f32),
    ))(sp4, g4, dinv, b4p, x1, x2, x3)

    key2d = keycol.reshape(NPAD // 128, 128)
    batch2d = jnp.pad(batch, (0, NPAD - N), constant_values=G)\
        .reshape(NPAD // 128, 128)
    idxmat = _tc_call(_tc_topk, jax.ShapeDtypeStruct((G, 32), jnp.int32))(
        key2d, batch2d)

    pooled = _sc_pool_gather(xc, idxmat.reshape(NPOOL))

    wp = jnp.pad(
        caps_W.transpose(2, 0, 1).reshape(caps_W.shape[2], 512),
        ((0, 128 - caps_W.shape[2]), (0, 0)))
    out = _tc_call(_tc_head, jax.ShapeDtypeStruct((G, 10), f32))(
        pooled, wp, fc1_W, fc1_b.reshape(1, 128), fc2_W,
        fc2_b.reshape(1, 10))
    return out


# speedup vs baseline: 14.5360x; 14.5360x over previous
import functools

import jax
import jax.numpy as jnp
from jax import lax
from jax.experimental import pallas as pl
from jax.experimental.pallas import tpu as pltpu
from jax.experimental.pallas import tpu_sc as plsc

N = 10000
E = 320000
G = 64
K = 30
NPAD = 10240
TRASH = N
DEGW = 16
CHUNK = 128
NCHUNKS = E // CHUNK
NC = 2
NS = 16
NW = NC * NS
RPT = NPAD // NS

@functools.lru_cache(maxsize=None)
def _mesh():
    return plsc.VectorSubcoreMesh(
        core_axis_name="c", subcore_axis_name="s",
        num_cores=NC, num_subcores=NS)


def _worker_ids():
    cid = lax.axis_index("c")
    sid = lax.axis_index("s")
    return cid, sid, sid * NC + cid


def _zero_tile_rows(zbuf, shared, sid, width):
    def zrow(i, _):
        for c0 in range(0, width, 16):
            zbuf[i, pl.ds(c0, 16)] = jnp.zeros((16,), jnp.float32)
        return 0
    lax.fori_loop(0, RPT, zrow, 0)
    pltpu.sync_copy(zbuf, shared.at[pl.ds(sid * RPT, RPT)])


def _edge_loop(wid, body):
    n_my = (NCHUNKS - wid + NW - 1) // NW

    def step(t, _):
        body(wid + t * NW)
        return 0
    lax.fori_loop(0, n_my, step, 0)


@functools.lru_cache(maxsize=None)
def _build_sc_prep():
    return functools.partial(
        pl.kernel,
        out_type=(jax.ShapeDtypeStruct((E,), jnp.int32),
                  jax.ShapeDtypeStruct((NC, NPAD, DEGW), jnp.float32)),
        mesh=_mesh(),
        scratch_types=[
            pltpu.VMEM((CHUNK,), jnp.int32),
            pltpu.VMEM((CHUNK,), jnp.int32),
            pltpu.VMEM((CHUNK,), jnp.int32),
            pltpu.VMEM((CHUNK, DEGW), jnp.float32),
            pltpu.VMEM((RPT, DEGW), jnp.float32),
            pltpu.VMEM_SHARED((NPAD, DEGW), jnp.float32),
        ],
        compiler_params=pltpu.CompilerParams(use_tc_tiling_on_sc=False),
    )(_sc_prep_body)


def _sc_prep_body(row_hbm, col_hbm, colp_hbm, degp_hbm,
                  r_v, c_v, cp_v, ones_v, zbuf, deg_sh):
    cid, sid, wid = _worker_ids()

    def fill_ones(i, _):
        ones_v[i, :] = jnp.ones((16,), jnp.float32)
        return 0
    lax.fori_loop(0, CHUNK, fill_ones, 0)
    _zero_tile_rows(zbuf, deg_sh, sid, DEGW)
    plsc.subcore_barrier()

    def do_chunk(j):
        base = j * CHUNK
        pltpu.sync_copy(row_hbm.at[pl.ds(base, CHUNK)], r_v)
        pltpu.sync_copy(col_hbm.at[pl.ds(base, CHUNK)], c_v)
        for i in range(0, CHUNK, 16):
            r16 = r_v[pl.ds(i, 16)]
            c16 = c_v[pl.ds(i, 16)]
            cp_v[pl.ds(i, 16)] = jnp.where(
                r16 != c16, c16, jnp.full((16,), TRASH, jnp.int32))
        pltpu.sync_copy(cp_v, colp_hbm.at[pl.ds(base, CHUNK)])
        pltpu.sync_copy(ones_v, deg_sh.at[cp_v], add=True)

    _edge_loop(wid, do_chunk)
    plsc.subcore_barrier()
    pltpu.sync_copy(deg_sh.at[pl.ds(sid * RPT, RPT)],
                    degp_hbm.at[cid, pl.ds(sid * RPT, RPT)])


@functools.lru_cache(maxsize=None)
def _make_sc_scatter(fw):
    @functools.partial(
        pl.kernel,
        out_type=jax.ShapeDtypeStruct((NC, NPAD, fw), jnp.float32),
        mesh=_mesh(),
        scratch_types=[
            pltpu.VMEM((CHUNK,), jnp.int32),
            pltpu.VMEM((CHUNK,), jnp.int32),
            pltpu.VMEM((CHUNK, fw), jnp.float32),
            pltpu.VMEM((RPT, fw), jnp.float32),
            pltpu.VMEM_SHARED((NPAD, fw), jnp.float32),
            pltpu.SemaphoreType.DMA,
        ],
        compiler_params=pltpu.CompilerParams(use_tc_tiling_on_sc=False),
    )
    def sc_scatter(g_hbm, row_hbm, colp_hbm, sp_hbm,
                   ridx_v, cidx_v, rows_v, zbuf, acc_sh, sem):
        cid, sid, wid = _worker_ids()
        _zero_tile_rows(zbuf, acc_sh, sid, fw)
        plsc.subcore_barrier()

        def do_chunk(j):
            base = j * CHUNK
            pltpu.sync_copy(row_hbm.at[pl.ds(base, CHUNK)], ridx_v)
            pltpu.sync_copy(colp_hbm.at[pl.ds(base, CHUNK)], cidx_v)
            pltpu.async_copy(g_hbm.at[ridx_v], rows_v, sem).wait()
            pltpu.sync_copy(rows_v, acc_sh.at[cidx_v], add=True)

        _edge_loop(wid, do_chunk)
        plsc.subcore_barrier()
        pltpu.sync_copy(acc_sh.at[pl.ds(sid * RPT, RPT)],
                        sp_hbm.at[cid, pl.ds(sid * RPT, RPT)])

    return sc_scatter


NPOOL = G * 32
PPW = NPOOL // NW


@functools.lru_cache(maxsize=None)
def _build_sc_pool_gather():
    return functools.partial(
        pl.kernel,
        out_type=jax.ShapeDtypeStruct((NPOOL, 128), jnp.float32),
        mesh=_mesh(),
        scratch_types=[
            pltpu.VMEM((PPW,), jnp.int32),
            pltpu.VMEM((PPW, 128), jnp.float32),
            pltpu.SemaphoreType.DMA,
        ],
        compiler_params=pltpu.CompilerParams(use_tc_tiling_on_sc=False),
    )(_sc_pool_gather_body)


def _sc_pool_gather_body(xc_hbm, idx_hbm, pooled_hbm, idx_v, rows_v, sem):
    _, _, wid = _worker_ids()
    base = wid * PPW
    pltpu.sync_copy(idx_hbm.at[pl.ds(base, PPW)], idx_v)
    pltpu.async_copy(xc_hbm.at[idx_v], rows_v, sem).wait()
    pltpu.sync_copy(rows_v, pooled_hbm.at[pl.ds(base, PPW)])


def _sc_prep(row, col):
    return _build_sc_prep()(row, col)


def _sc_scatter(fw, g, row, colp):
    return _make_sc_scatter(fw)(g, row, colp)


def _sc_pool_gather(xc, idx):
    return _build_sc_pool_gather()(xc, idx)


def _tc_first(x_ref, w1_ref, degp_ref, dinv_ref, g1_ref):
    deg = degp_ref[0, :, 0:1] + degp_ref[1, :, 0:1]
    dinv = lax.rsqrt(deg[0:N] + 1.0)
    dinv_ref[...] = dinv
    h = jnp.dot(x_ref[...], w1_ref[...], preferred_element_type=jnp.float32)
    g1_ref[0:N, :] = h * dinv
    g1_ref[N:NPAD, :] = jnp.zeros((NPAD - N, 32), jnp.float32)


def _tc_combine(sp_ref, g_ref, dinv_ref, b_ref, wn_ref, xl_ref, gn_ref):
    s = sp_ref[0, 0:N, :] + sp_ref[1, 0:N, :]
    dinv = dinv_ref[...]
    xl = jnp.tanh(dinv * (s + g_ref[0:N, :]) + b_ref[...])
    xl_ref[...] = xl
    h = jnp.dot(xl, wn_ref[...], preferred_element_type=jnp.float32)
    fw = gn_ref.shape[1]
    gn_ref[0:N, :] = h * dinv
    gn_ref[N:NPAD, :] = jnp.zeros((NPAD - N, fw), jnp.float32)


def _tc_last(sp_ref, g_ref, dinv_ref, b_ref, x1_ref, x2_ref, x3_ref,
             xc_ref):
    s = sp_ref[0, 0:N, :] + sp_ref[1, 0:N, :]
    x4 = jnp.tanh(dinv_ref[...] * (s + g_ref[0:N, :]) + b_ref[...])
    xc_ref[0:N, 0:32] = x1_ref[...]
    xc_ref[0:N, 32:64] = x2_ref[...]
    xc_ref[0:N, 64:96] = x3_ref[...]
    xc_ref[0:N, 96:112] = x4
    xc_ref[0:N, 112:128] = jnp.zeros((N, 16), jnp.float32)
    xc_ref[N:NPAD, :] = jnp.zeros((NPAD - N, 128), jnp.float32)


def _tc_topk(key_ref, batch_ref, idx_ref):
    key0 = key_ref[...]
    batch2d = batch_ref[...]
    gids = lax.broadcasted_iota(jnp.int32, (G, 1, 1), 0)
    gmask = batch2d[None, :, :] == gids
    iota_n = (lax.broadcasted_iota(jnp.int32, (NPAD // 128, 128), 0) * 128
              + lax.broadcasted_iota(jnp.int32, (NPAD // 128, 128), 1))
    lane32 = lax.broadcasted_iota(jnp.int32, (G, 32), 1)
    big = jnp.int32(2**30)

    def step(j, carry):
        key, idxmat = carry
        kv = jnp.where(gmask, key[None, :, :], -2.0)
        maxv = jnp.max(kv, axis=(1, 2))
        eq = kv == maxv[:, None, None]
        idxv = jnp.min(jnp.where(eq, iota_n[None, :, :], big), axis=(1, 2))
        idxv = jnp.where(maxv < -1.5, TRASH, idxv)
        idxmat = jnp.where(lane32 == j, idxv[:, None], idxmat)
        sel = jnp.any(iota_n[None, :, :] == idxv[:, None, None], axis=0)
        key = jnp.where(sel, -2.0, key)
        return key, idxmat

    idx0 = jnp.full((G, 32), TRASH, jnp.int32)
    _, idxmat = lax.fori_loop(0, K, step, (key0, idx0))
    idx_ref[...] = idxmat


def _tc_head(pooled_ref, wp_ref, fc1w_ref, fc1b_ref, fc2w_ref, fc2b_ref,
             out_ref):
    P = jnp.dot(pooled_ref[...], wp_ref[...],
                preferred_element_type=jnp.float32)
    bd = (lax.broadcasted_iota(jnp.int32, (512, 16), 0) // 32
          == lax.broadcasted_iota(jnp.int32, (512, 16), 1)
          ).astype(jnp.float32)
    n1 = jnp.dot(P * P, bd, preferred_element_type=jnp.float32)
    slot_ok = (lax.broadcasted_iota(jnp.int32, (NPOOL, 16), 0) % 32) < K
    out = jnp.sum(P.reshape(G, 32, 512), axis=1) * (1.0 / K)

    for _ in range(3):
        orows = jnp.broadcast_to(
            out.reshape(G, 1, 512), (G, 32, 512)).reshape(NPOOL, 512)
        dot = jnp.dot(P * orows, bd, preferred_element_type=jnp.float32)
        n2 = jnp.dot(out * out, bd, preferred_element_type=jnp.float32)
        n2r = jnp.broadcast_to(
            n2.reshape(G, 1, 16), (G, 32, 16)).reshape(NPOOL, 16)
        logits = dot / jnp.maximum(n1 + n2r - dot, 1e-8)
        logits = jnp.where(slot_ok, logits, -1e30)
        l3 = logits.reshape(G, 32, 16)
        m = jnp.max(l3, axis=1, keepdims=True)
        e = jnp.exp(l3 - m)
        coef = (e / jnp.sum(e, axis=1, keepdims=True)).reshape(NPOOL, 16)
        cexp = jnp.broadcast_to(
            coef.reshape(NPOOL, 16, 1), (NPOOL, 16, 32)).reshape(NPOOL, 512)
        out = jnp.sum((cexp * P).reshape(G, 32, 512), axis=1)

    n2f = jnp.dot(out * out, bd, preferred_element_type=jnp.float32)
    scale = (n2f / (1.0 + n2f)) / jnp.sqrt(n2f + 1e-8)
    sexp = jnp.broadcast_to(
        scale.reshape(G, 16, 1), (G, 16, 32)).reshape(G, 512)
    caps = out * sexp
    h = jnp.maximum(
        jnp.dot(caps, fc1w_ref[...], preferred_element_type=jnp.float32)
        + fc1b_ref[...], 0.0)
    z = (jnp.dot(h, fc2w_ref[...], preferred_element_type=jnp.float32)
         + fc2b_ref[...])
    m = jnp.max(z, axis=1, keepdims=True)
    lse = m + jnp.log(jnp.sum(jnp.exp(z - m), axis=1, keepdims=True))
    out_ref[...] = z - lse


def _tc_call(body, out_shapes):
    return pl.pallas_call(body, out_shape=out_shapes)


def kernel(x, edge_index, batch, W1, b1, W2, b2, W3, b3, W4, b4,
           caps_W, fc1_W, fc1_b, fc2_W, fc2_b):
    f32 = jnp.float32
    row = edge_index[0]
    col = edge_index[1]

    colp, degp = _sc_prep(row, col)

    dinv, g1 = _tc_call(_tc_first, (
        jax.ShapeDtypeStruct((N, 1), f32),
        jax.ShapeDtypeStruct((NPAD, 32), f32),
    ))(x, W1, degp)

    sp1 = _sc_scatter(32, g1, row, colp)
    x1, g2 = _tc_call(_tc_combine, (
        jax.ShapeDtypeStruct((N, 32), f32),
        jax.ShapeDtypeStruct((NPAD, 32), f32),
    ))(sp1, g1, dinv, b1.reshape(1, 32), W2)

    sp2 = _sc_scatter(32, g2, row, colp)
    x2, g3 = _tc_call(_tc_combine, (
        jax.ShapeDtypeStruct((N, 32), f32),
        jax.ShapeDtypeStruct((NPAD, 32), f32),
    ))(sp2, g2, dinv, b2.reshape(1, 32), W3)

    sp3 = _sc_scatter(32, g3, row, colp)
    W4p = jnp.pad(W4, ((0, 0), (0, 15)))
    x3, g4 = _tc_call(_tc_combine, (
        jax.ShapeDtypeStruct((N, 32), f32),
        jax.ShapeDtypeStruct((NPAD, 16), f32),
    ))(sp3, g3, dinv, b3.reshape(1, 32), W4p)

    sp4 = _sc_scatter(16, g4, row, colp)
    b4p = jnp.pad(b4, (0, 15)).reshape(1, 16)
    xc = _tc_call(_tc_last, jax.ShapeDtypeStruct((NPAD, 128), f32))(
        sp4, g4, dinv, b4p, x1, x2, x3)

    key2d = xc[:, 96].reshape(NPAD // 128, 128)
    batch2d = jnp.pad(batch, (0, NPAD - N), constant_values=G)\
        .reshape(NPAD // 128, 128)
    idxmat = _tc_call(_tc_topk, jax.ShapeDtypeStruct((G, 32), jnp.int32))(
        key2d, batch2d)

    pooled = _sc_pool_gather(xc, idxmat.reshape(NPOOL))

    wp = jnp.pad(
        caps_W.transpose(2, 0, 1).reshape(caps_W.shape[2], 512),
        ((0, 128 - caps_W.shape[2]), (0, 0)))
    out = _tc_call(_tc_head, jax.ShapeDtypeStruct((G, 10), f32))(
        pooled, wp, fc1_W, fc1_b.reshape(1, 128), fc2_W,
        fc2_b.reshape(1, 10))
    return out
